# baseline (speedup 1.0000x reference)
import functools

import jax
import jax.numpy as jnp
from jax import lax
from jax.experimental import pallas as pl
from jax.experimental.pallas import tpu as pltpu
from jax.experimental.pallas import tpu_sc as plsc

NC = 2
NS = 16
L = 16
NW = NC * NS

D = 128
ROWS = 4096 * 200
ROWS_PER_W = ROWS // NW
CHUNK = 160
NCHUNK = ROWS_PER_W // CHUNK
CB = CHUNK * D
NBUF = 2


def _make_sc_permute():
  mesh = plsc.VectorSubcoreMesh(core_axis_name="c", subcore_axis_name="s")

  @functools.partial(
      pl.kernel,
      mesh=mesh,
      out_type=jax.ShapeDtypeStruct((ROWS * D,), jnp.float32),
      scratch_types=(
          [pltpu.VMEM((CB,), jnp.float32) for _ in range(2 * NBUF)]
          + [pltpu.VMEM((D,), jnp.int32)]
          + [pltpu.SemaphoreType.DMA for _ in range(2 * NBUF)]
      ),
      compiler_params=pltpu.CompilerParams(needs_layout_passes=False),
  )
  def permute_kernel(x_hbm, perm_hbm, out_hbm, *scratch):
    ibufs = scratch[:NBUF]
    obufs = scratch[NBUF:2 * NBUF]
    permb = scratch[2 * NBUF]
    isems = scratch[2 * NBUF + 1:2 * NBUF + 1 + NBUF]
    osems = scratch[2 * NBUF + 1 + NBUF:]

    wid = lax.axis_index("s") * NC + lax.axis_index("c")

    pltpu.sync_copy(perm_hbm, permb)
    perm_vecs = [permb[pl.ds(c * L, L)] for c in range(D // L)]

    def permute_chunk(ib, ob):
      @plsc.parallel_loop(0, CHUNK, unroll=4)
      def row_body(r):
        rb = r * D
        for c in range(D // L):
          ob[pl.ds(rb + c * L, L)] = plsc.load_gather(ib, [perm_vecs[c] + rb])

    def chunk_off(g):
      return (g * NW + wid) * CB

    def start_in(g, b):
      pltpu.async_copy(x_hbm.at[pl.ds(chunk_off(g), CB)], ibufs[b], isems[b])

    def start_out(g, b):
      pltpu.async_copy(obufs[b], out_hbm.at[pl.ds(chunk_off(g), CB)],
                       osems[b])

    def wait_in(b):
      pltpu.make_async_copy(x_hbm.at[pl.ds(wid * CB, CB)], ibufs[b],
                            isems[b]).wait()

    def wait_out(b):
      pltpu.make_async_copy(obufs[b], out_hbm.at[pl.ds(wid * CB, CB)],
                            osems[b]).wait()

    for b in range(NBUF):
      start_in(b, b)

    def ring_body(i, carry):
      g = i * NBUF
      for b in range(NBUF):
        @pl.when(i > 0)
        def _():
          wait_out(b)

        wait_in(b)
        permute_chunk(ibufs[b], obufs[b])
        start_out(g + b, b)

        @pl.when(g + b + NBUF < NCHUNK)
        def _():
          start_in(g + b + NBUF, b)
      return carry

    lax.fori_loop(0, NCHUNK // NBUF, ring_body, 0)
    for b in range(NBUF):
      wait_out(b)

  return permute_kernel


_sc_permute = _make_sc_permute()


def kernel(input, permutation):
  x_flat = input.reshape(ROWS * D)
  out_flat = _sc_permute(x_flat, permutation)
  return out_flat.reshape(input.shape)


# speedup vs baseline: 1.0125x; 1.0125x over previous
import functools

import jax
import jax.numpy as jnp
from jax import lax
from jax.experimental import pallas as pl
from jax.experimental.pallas import tpu as pltpu
from jax.experimental.pallas import tpu_sc as plsc

NC = 2
NS = 16
L = 16
NW = NC * NS

D = 128
ROWS = 4096 * 200
ROWS_PER_W = ROWS // NW
CHUNK = 200
NCHUNK = ROWS_PER_W // CHUNK
CB = CHUNK * D
NBUF = 2


def _make_sc_permute():
  mesh = plsc.VectorSubcoreMesh(core_axis_name="c", subcore_axis_name="s")

  @functools.partial(
      pl.kernel,
      mesh=mesh,
      out_type=jax.ShapeDtypeStruct((ROWS * D,), jnp.float32),
      scratch_types=(
          [pltpu.VMEM((CB,), jnp.float32) for _ in range(2 * NBUF)]
          + [pltpu.VMEM((D,), jnp.int32)]
          + [pltpu.SemaphoreType.DMA for _ in range(2 * NBUF)]
      ),
      compiler_params=pltpu.CompilerParams(needs_layout_passes=False),
  )
  def permute_kernel(x_hbm, perm_hbm, out_hbm, *scratch):
    ibufs = scratch[:NBUF]
    obufs = scratch[NBUF:2 * NBUF]
    permb = scratch[2 * NBUF]
    isems = scratch[2 * NBUF + 1:2 * NBUF + 1 + NBUF]
    osems = scratch[2 * NBUF + 1 + NBUF:]

    wid = lax.axis_index("s") * NC + lax.axis_index("c")

    pltpu.sync_copy(perm_hbm, permb)
    perm_vecs = [permb[pl.ds(c * L, L)] for c in range(D // L)]

    def permute_chunk(ib, ob):
      @plsc.parallel_loop(0, CHUNK, unroll=4)
      def row_body(r):
        rb = r * D
        for c in range(D // L):
          ob[pl.ds(rb + c * L, L)] = plsc.load_gather(ib, [perm_vecs[c] + rb])

    def chunk_off(g):
      return (g * NW + wid) * CB

    def start_in(g, b):
      pltpu.async_copy(x_hbm.at[pl.ds(chunk_off(g), CB)], ibufs[b], isems[b])

    def start_out(g, b):
      pltpu.async_copy(obufs[b], out_hbm.at[pl.ds(chunk_off(g), CB)],
                       osems[b])

    def wait_in(b):
      pltpu.make_async_copy(x_hbm.at[pl.ds(wid * CB, CB)], ibufs[b],
                            isems[b]).wait()

    def wait_out(b):
      pltpu.make_async_copy(obufs[b], out_hbm.at[pl.ds(wid * CB, CB)],
                            osems[b]).wait()

    for b in range(NBUF):
      start_in(b, b)

    def ring_body(i, carry):
      g = i * NBUF
      for b in range(NBUF):
        @pl.when(i > 0)
        def _():
          wait_out(b)

        wait_in(b)
        permute_chunk(ibufs[b], obufs[b])
        start_out(g + b, b)

        @pl.when(g + b + NBUF < NCHUNK)
        def _():
          start_in(g + b + NBUF, b)
      return carry

    lax.fori_loop(0, NCHUNK // NBUF, ring_body, 0)
    for b in range(NBUF):
      wait_out(b)

  return permute_kernel


_sc_permute = _make_sc_permute()


def kernel(input, permutation):
  x_flat = input.reshape(ROWS * D)
  out_flat = _sc_permute(x_flat, permutation)
  return out_flat.reshape(input.shape)
